# baseline (speedup 1.0000x reference)
import functools
import jax
import jax.numpy as jnp
from jax import lax
from jax.experimental import pallas as pl
from jax.experimental.pallas import tpu as pltpu, tpu_sc as plsc

D_MODEL = 1024
SEQ_LEN = 2048
BATCH = 4
SCALE = 32.0

NUM_CORES = 2
NUM_SUBCORES = 16
NUM_WORKERS = NUM_CORES * NUM_SUBCORES
S_PER_W = SEQ_LEN // NUM_WORKERS
CHUNK_P = 8
CHUNKS = S_PER_W // CHUNK_P
ROWS = BATCH * CHUNK_P
NBUF = 3
LANES = 16
NSLICE = D_MODEL // LANES
IDS = BATCH * S_PER_W

_mesh = plsc.VectorSubcoreMesh(core_axis_name="c", subcore_axis_name="s")


@functools.partial(
    pl.kernel,
    mesh=_mesh,
    out_type=[
        jax.ShapeDtypeStruct((BATCH - 1, SEQ_LEN, D_MODEL), jnp.float32),
        jax.ShapeDtypeStruct((1, SEQ_LEN, D_MODEL), jnp.float32),
    ],
    scratch_types=[
        pltpu.VMEM((IDS,), jnp.int32),
        pltpu.VMEM((IDS,), jnp.int32),
        pltpu.VMEM((2, CHUNK_P, D_MODEL), jnp.float32),
        pltpu.VMEM((NBUF, ROWS, D_MODEL), jnp.float32),
        pltpu.SemaphoreType.DMA,
        pltpu.SemaphoreType.DMA,
        pltpu.SemaphoreType.DMA,
    ],
)
def _embed(
    x_hbm, table_hbm, pe_hbm, out_hbm, out1_hbm, idx_n, idx_p, pe_v, rows_v, gsem, ssem, psem
):
    wid = lax.axis_index("s") * NUM_CORES + lax.axis_index("c")
    s_base = wid * S_PER_W

    idx_handles = [
        pltpu.async_copy(
            x_hbm.at[b, pl.ds(s_base, S_PER_W)],
            idx_n.at[pl.ds(b * S_PER_W, S_PER_W)],
            psem,
        )
        for b in range(BATCH)
    ]
    for h in idx_handles:
        h.wait()

    low = lax.iota(jnp.int32, LANES) < 8

    def permute_chunk(g):
        for b0 in range(0, BATCH, 2):
            v1 = idx_n[pl.ds(b0 * S_PER_W + g * CHUNK_P, LANES)]
            v2 = idx_n[pl.ds((b0 + 1) * S_PER_W + g * CHUNK_P - 8, LANES)]
            idx_p[pl.ds(g * ROWS + b0 * CHUNK_P, LANES)] = jnp.where(low, v1, v2)

    def issue_gather(g):
        off = pl.multiple_of(g * ROWS, 8)
        pltpu.async_copy(
            table_hbm.at[idx_p.at[pl.ds(off, ROWS)]], rows_v.at[g % NBUF], gsem
        )

    def issue_pe(g):
        pltpu.async_copy(
            pe_hbm.at[0, pl.ds(s_base + g * CHUNK_P, CHUNK_P)], pe_v.at[g % 2], psem
        )

    def wait_gather():
        pltpu.make_async_copy(table_hbm.at[pl.ds(0, ROWS)], rows_v.at[0], gsem).wait()

    def wait_pe():
        pltpu.make_async_copy(pe_hbm.at[0, pl.ds(0, CHUNK_P)], pe_v.at[0], psem).wait()

    def wait_store():
        pltpu.make_async_copy(rows_v.at[0], out_hbm.at[0, pl.ds(0, ROWS)], ssem).wait()

    permute_chunk(0)
    issue_gather(0)
    issue_pe(0)
    for g in range(1, CHUNKS):
        permute_chunk(g)

    def body(g, carry):
        buf = g % NBUF
        pebuf = g % 2

        @pl.when(g + 1 < CHUNKS)
        def _issue_next():
            @pl.when(g >= 2)
            def _drain():
                wait_store()

            issue_gather(g + 1)
            issue_pe(g + 1)

        wait_gather()
        wait_pe()

        @plsc.parallel_loop(0, CHUNK_P)
        def _p_loop(p):
            @plsc.parallel_loop(0, NSLICE, unroll=4)
            def _j_loop(j):
                c = j * LANES
                pe_vec = pe_v[pebuf, p, pl.ds(c, LANES)]
                for b in range(BATCH):
                    r = b * CHUNK_P + p
                    rows_v[buf, r, pl.ds(c, LANES)] = (
                        rows_v[buf, r, pl.ds(c, LANES)] * SCALE + pe_vec
                    )

        for b in range(BATCH):
            pltpu.async_copy(
                rows_v.at[buf, pl.ds(b * CHUNK_P, CHUNK_P)],
                out_hbm.at[b, pl.ds(s_base + g * CHUNK_P, CHUNK_P)]
                if b < BATCH - 1
                else out1_hbm.at[0, pl.ds(s_base + g * CHUNK_P, CHUNK_P)],
                ssem,
            )

        return carry

    lax.fori_loop(0, CHUNKS, body, 0)
    wait_store()
    wait_store()
    wait_store()


def kernel(x, token_table, pe):
    out0, out1 = _embed(x.astype(jnp.int32), token_table, pe)
    return jnp.concatenate([out0, out1], axis=0)


# speedup vs baseline: 1.4864x; 1.4864x over previous
import functools
import jax
import jax.numpy as jnp
from jax import lax
from jax.experimental import pallas as pl
from jax.experimental.pallas import tpu as pltpu, tpu_sc as plsc

D_MODEL = 1024
SEQ_LEN = 2048
BATCH = 4
SCALE = 32.0

NUM_CORES = 2
NUM_SUBCORES = 16
NUM_WORKERS = NUM_CORES * NUM_SUBCORES
S_PER_W = SEQ_LEN // NUM_WORKERS
CHUNK_P = 8
CHUNKS = S_PER_W // CHUNK_P
ROWS = BATCH * CHUNK_P
NBUF = 3
LANES = 16
NSLICE = D_MODEL // LANES
IDS = BATCH * S_PER_W

_mesh = plsc.VectorSubcoreMesh(core_axis_name="c", subcore_axis_name="s")


@functools.partial(
    pl.kernel,
    mesh=_mesh,
    out_type=jax.ShapeDtypeStruct((BATCH, SEQ_LEN, D_MODEL), jnp.float32),
    scratch_types=[
        pltpu.VMEM((IDS,), jnp.int32),
        pltpu.VMEM((IDS,), jnp.int32),
        pltpu.VMEM((2, CHUNK_P, D_MODEL), jnp.float32),
        pltpu.VMEM((NBUF, ROWS, D_MODEL), jnp.float32),
        pltpu.SemaphoreType.DMA,
        pltpu.SemaphoreType.DMA,
        pltpu.SemaphoreType.DMA,
    ],
)
def _embed(
    x_hbm, table_hbm, pe_hbm, out_hbm, idx_n, idx_p, pe_v, rows_v, gsem, ssem, psem
):
    wid = lax.axis_index("s") * NUM_CORES + lax.axis_index("c")
    s_base = wid * S_PER_W

    idx_handles = [
        pltpu.async_copy(
            x_hbm.at[b, pl.ds(s_base, S_PER_W)],
            idx_n.at[pl.ds(b * S_PER_W, S_PER_W)],
            psem,
        )
        for b in range(BATCH)
    ]
    for h in idx_handles:
        h.wait()

    low = lax.iota(jnp.int32, LANES) < 8

    def permute_chunk(g):
        for b0 in range(0, BATCH, 2):
            v1 = idx_n[pl.ds(b0 * S_PER_W + g * CHUNK_P, LANES)]
            v2 = idx_n[pl.ds((b0 + 1) * S_PER_W + g * CHUNK_P - 8, LANES)]
            idx_p[pl.ds(g * ROWS + b0 * CHUNK_P, LANES)] = jnp.where(low, v1, v2)

    def issue_gather(g):
        off = pl.multiple_of(g * ROWS, 8)
        pltpu.async_copy(
            table_hbm.at[idx_p.at[pl.ds(off, ROWS)]], rows_v.at[g % NBUF], gsem
        )

    def issue_pe(g):
        pltpu.async_copy(
            pe_hbm.at[0, pl.ds(s_base + g * CHUNK_P, CHUNK_P)], pe_v.at[g % 2], psem
        )

    def wait_gather():
        pltpu.make_async_copy(table_hbm.at[pl.ds(0, ROWS)], rows_v.at[0], gsem).wait()

    def wait_pe():
        pltpu.make_async_copy(pe_hbm.at[0, pl.ds(0, CHUNK_P)], pe_v.at[0], psem).wait()

    def wait_store():
        pltpu.make_async_copy(rows_v.at[0], out_hbm.at[0, pl.ds(0, ROWS)], ssem).wait()

    permute_chunk(0)
    issue_gather(0)
    issue_pe(0)
    for g in range(1, CHUNKS):
        permute_chunk(g)

    def body(g, carry):
        buf = g % NBUF
        pebuf = g % 2

        @pl.when(g + 1 < CHUNKS)
        def _issue_next():
            @pl.when(g >= 2)
            def _drain():
                wait_store()

            issue_gather(g + 1)
            issue_pe(g + 1)

        wait_gather()
        wait_pe()

        @plsc.parallel_loop(0, CHUNK_P)
        def _p_loop(p):
            @plsc.parallel_loop(0, NSLICE, unroll=4)
            def _j_loop(j):
                c = j * LANES
                pe_vec = pe_v[pebuf, p, pl.ds(c, LANES)]
                for b in range(BATCH):
                    r = b * CHUNK_P + p
                    rows_v[buf, r, pl.ds(c, LANES)] = (
                        rows_v[buf, r, pl.ds(c, LANES)] * SCALE + pe_vec
                    )

        for b in range(BATCH):
            pltpu.async_copy(
                rows_v.at[buf, pl.ds(b * CHUNK_P, CHUNK_P)],
                out_hbm.at[b, pl.ds(s_base + g * CHUNK_P, CHUNK_P)],
                ssem,
            )

        return carry

    lax.fori_loop(0, CHUNKS, body, 0)
    wait_store()
    wait_store()
    wait_store()


def kernel(x, token_table, pe):
    return _embed(x.astype(jnp.int32), token_table, pe)
